# baseline (speedup 1.0000x reference)
import functools

import jax
import jax.numpy as jnp
from jax import lax
from jax.experimental import pallas as pl
from jax.experimental.pallas import tpu as pltpu
from jax.experimental.pallas import tpu_sc as plsc

NSEG = 16384
N_ATOMS = 1048576
EMB = 64
NZ = 119
ZPAD = 128
NC, NS, L = 2, 16, 16
NW = NC * NS
CHUNK = N_ATOMS // NW
NVEC = CHUNK // L
BLK = 512


UNROLL = 16


def _sc_body(att_h, w_h, sae_h, num_h, mol_h, out_h,
             att_vm, w_vm, sae_vm, ctab_vm, nums_vm, mols_vm,
             acc_vm, idx_vm, shared, sem_n, sem_m):
    c = lax.axis_index("c")
    s = lax.axis_index("s")
    base = (s * NC + c) * CHUNK

    cp_n = pltpu.make_async_copy(num_h.at[pl.ds(base, CHUNK)], nums_vm, sem_n)
    cp_m = pltpu.make_async_copy(mol_h.at[pl.ds(base, CHUNK)],
                                 mols_vm.at[pl.ds(0, CHUNK)], sem_m)
    cp_n.start()
    cp_m.start()

    pltpu.sync_copy(att_h, att_vm)
    pltpu.sync_copy(w_h, w_vm)
    pltpu.sync_copy(sae_h, sae_vm)

    accs = [jnp.zeros((L,), jnp.float32) for _ in range(ZPAD // L)]
    for db in range(EMB // L):
        wv = w_vm[pl.ds(db * L, L)]
        for j in range(L):
            ws = wv[j]
            d = db * L + j
            for zb in range(ZPAD // L):
                accs[zb] = accs[zb] + att_vm[d, pl.ds(zb * L, L)] * ws
    for zb in range(ZPAD // L):
        ctab_vm[pl.ds(zb * L, L)] = accs[zb] + sae_vm[pl.ds(zb * L, L)]

    zero16 = jnp.zeros((L,), jnp.float32)

    @plsc.parallel_loop(0, NSEG // L, unroll=UNROLL)
    def _(i):
        acc_vm[pl.ds(i * L, L)] = zero16

    @pl.when(s == 0)
    def _():
        pltpu.sync_copy(acc_vm.at[pl.ds(0, NSEG)], shared)

    cp_m.wait()
    s_lo = jnp.min(mols_vm[pl.ds(0, L)])
    s_hi = jnp.max(mols_vm[pl.ds(CHUNK - L, L)])
    lo = (s_lo // BLK) * BLK
    nblk = (s_hi - lo) // BLK + 1

    cp_n.wait()

    mols0 = mols_vm[pl.ds(0, L)]
    zrun = jnp.zeros((L,), jnp.float32)

    @plsc.parallel_loop(0, NVEC, unroll=UNROLL, carry=(zrun, mols0))
    def _(i, carry):
        S, cur = carry
        o = i * L
        nums = nums_vm[pl.ds(o, L)]
        mols = mols_vm[pl.ds(o, L)]
        vals = plsc.load_gather(ctab_vm, [nums])
        changed = cur != mols
        plsc.addupdate_scatter(acc_vm, [cur], S, mask=changed)
        S = jnp.where(changed, vals, S + vals)
        return S, mols
    S_fin, cur_fin = _
    plsc.addupdate_scatter(acc_vm, [cur_fin], S_fin)

    plsc.subcore_barrier()
    iota16 = lax.iota(jnp.int32, L)

    def cbody(j, carry):
        bj = lo + j * BLK
        for m in range(BLK // L):
            idx_vm[pl.ds(m * L, L)] = bj + m * L + iota16
        pltpu.sync_copy(acc_vm.at[pl.ds(bj, BLK)], shared.at[idx_vm], add=True)
        return carry
    lax.fori_loop(0, nblk, cbody, 0)

    plsc.subcore_barrier()

    @pl.when(s == 0)
    def _():
        pltpu.sync_copy(shared, out_h.at[c])


@functools.partial(jax.jit, static_argnames=("interpret",))
def _sc_call(att, w, sae, numbers, mol_idx, interpret=False):
    mesh = plsc.VectorSubcoreMesh(core_axis_name="c", subcore_axis_name="s",
                                  num_cores=NC, num_subcores=NS)
    f = pl.kernel(
        _sc_body,
        out_type=jax.ShapeDtypeStruct((NC, NSEG), jnp.float32),
        mesh=mesh,
        scratch_types=[
            pltpu.VMEM((EMB, ZPAD), jnp.float32),
            pltpu.VMEM((EMB,), jnp.float32),
            pltpu.VMEM((ZPAD,), jnp.float32),
            pltpu.VMEM((ZPAD,), jnp.float32),
            pltpu.VMEM((CHUNK,), jnp.int32),
            pltpu.VMEM((CHUNK + L,), jnp.int32),
            pltpu.VMEM((NSEG + L,), jnp.float32),
            pltpu.VMEM((BLK,), jnp.int32),
            pltpu.VMEM_SHARED((NSEG,), jnp.float32),
            pltpu.SemaphoreType.DMA,
            pltpu.SemaphoreType.DMA,
        ],
        compiler_params=pltpu.CompilerParams(needs_layout_passes=False),
        interpret=interpret,
    )
    return f(att, w, sae, numbers, mol_idx)


def kernel(numbers, mol_idx, charge, atom_table, w, sae_tensor):
    del charge
    att = jnp.zeros((EMB, ZPAD), jnp.float32).at[:, :NZ].set(atom_table.T)
    sae = sae_tensor[:ZPAD]
    parts = _sc_call(att, w, sae, numbers, mol_idx)
    return parts[0] + parts[1]


# speedup vs baseline: 1.1307x; 1.1307x over previous
import functools

import jax
import jax.numpy as jnp
from jax import lax
from jax.experimental import pallas as pl
from jax.experimental.pallas import tpu as pltpu
from jax.experimental.pallas import tpu_sc as plsc

NSEG = 16384
N_ATOMS = 1048576
EMB = 64
NZ = 119
ZPAD = 128
NC, NS, L = 2, 16, 16
NW = NC * NS
CHUNK = N_ATOMS // NW
NVEC = CHUNK // L
BLK = 512


UNROLL = 16


def _sc_body(att_h, w_h, sae_h, num_h, mol_h, out_h,
             att_vm, w_vm, sae_vm, ctab_vm, nums_vm, mols_vm,
             acc_vm, idx_vm, shared, sem_n, sem_m):
    c = lax.axis_index("c")
    s = lax.axis_index("s")
    base = (s * NC + c) * CHUNK

    cp_n = pltpu.make_async_copy(num_h.at[pl.ds(base, CHUNK)], nums_vm, sem_n)
    cp_m = pltpu.make_async_copy(mol_h.at[pl.ds(base, CHUNK)],
                                 mols_vm.at[pl.ds(0, CHUNK)], sem_m)
    cp_n.start()
    cp_m.start()

    pltpu.sync_copy(att_h, att_vm)
    pltpu.sync_copy(w_h, w_vm)
    pltpu.sync_copy(sae_h, sae_vm)

    accs = [jnp.zeros((L,), jnp.float32) for _ in range(ZPAD // L)]
    for db in range(EMB // L):
        wv = w_vm[pl.ds(db * L, L)]
        for j in range(L):
            ws = wv[j]
            d = db * L + j
            for zb in range(ZPAD // L):
                accs[zb] = accs[zb] + att_vm[d, pl.ds(zb * L, L)] * ws
    for zb in range(ZPAD // L):
        v = accs[zb] + sae_vm[pl.ds(zb * L, L)]
        for j in range(L):
            ctab_vm[pl.ds((zb * L + j) * L, L)] = jnp.broadcast_to(v[j], (L,))

    zero16 = jnp.zeros((L,), jnp.float32)

    @plsc.parallel_loop(0, NSEG // L, unroll=UNROLL)
    def _(i):
        acc_vm[pl.ds(i * L, L)] = zero16

    @pl.when(s == 0)
    def _():
        pltpu.sync_copy(acc_vm.at[pl.ds(0, NSEG)], shared)

    cp_m.wait()
    mols_vm[pl.ds(CHUNK, L)] = jnp.full((L,), NSEG, jnp.int32)
    s_lo = jnp.min(mols_vm[pl.ds(0, L)])
    s_hi = jnp.max(mols_vm[pl.ds(CHUNK - L, L)])
    lo = (s_lo // BLK) * BLK
    nblk = (s_hi - lo) // BLK + 1

    cp_n.wait()

    iota16 = lax.iota(jnp.int32, L)

    @plsc.parallel_loop(0, NVEC, unroll=UNROLL,
                        carry=jnp.zeros((L,), jnp.float32))
    def _(i, run):
        o = i * L
        nums = nums_vm[pl.ds(o, L)]
        mols = mols_vm[pl.ds(o, L)]
        moln = mols_vm[pl.ds(o + 1, L)]
        vals = plsc.load_gather(ctab_vm, [nums * L + iota16])
        p = plsc.cumsum(vals)
        cum = p + run
        m = mols != moln
        plsc.addupdate_scatter(acc_vm, [mols], cum, mask=m)
        plsc.addupdate_scatter(acc_vm, [moln], -cum, mask=m)
        return run + jnp.broadcast_to(p[L - 1], (L,))

    plsc.subcore_barrier()

    def cbody(j, carry):
        bj = lo + j * BLK
        for m in range(BLK // L):
            idx_vm[pl.ds(m * L, L)] = bj + m * L + iota16
        pltpu.sync_copy(acc_vm.at[pl.ds(bj, BLK)], shared.at[idx_vm], add=True)
        return carry
    lax.fori_loop(0, nblk, cbody, 0)

    plsc.subcore_barrier()

    @pl.when(s == 0)
    def _():
        pltpu.sync_copy(shared, out_h.at[c])


@functools.partial(jax.jit, static_argnames=("interpret",))
def _sc_call(att, w, sae, numbers, mol_idx, interpret=False):
    mesh = plsc.VectorSubcoreMesh(core_axis_name="c", subcore_axis_name="s",
                                  num_cores=NC, num_subcores=NS)
    f = pl.kernel(
        _sc_body,
        out_type=jax.ShapeDtypeStruct((NC, NSEG), jnp.float32),
        mesh=mesh,
        scratch_types=[
            pltpu.VMEM((EMB, ZPAD), jnp.float32),
            pltpu.VMEM((EMB,), jnp.float32),
            pltpu.VMEM((ZPAD,), jnp.float32),
            pltpu.VMEM((ZPAD * L,), jnp.float32),
            pltpu.VMEM((CHUNK,), jnp.int32),
            pltpu.VMEM((CHUNK + L,), jnp.int32),
            pltpu.VMEM((NSEG + L,), jnp.float32),
            pltpu.VMEM((BLK,), jnp.int32),
            pltpu.VMEM_SHARED((NSEG,), jnp.float32),
            pltpu.SemaphoreType.DMA,
            pltpu.SemaphoreType.DMA,
        ],
        compiler_params=pltpu.CompilerParams(needs_layout_passes=False),
        interpret=interpret,
    )
    return f(att, w, sae, numbers, mol_idx)


def kernel(numbers, mol_idx, charge, atom_table, w, sae_tensor):
    del charge
    att = jnp.zeros((EMB, ZPAD), jnp.float32).at[:, :NZ].set(atom_table.T)
    sae = sae_tensor[:ZPAD]
    parts = _sc_call(att, w, sae, numbers, mol_idx)
    return parts[0] + parts[1]
